# baseline (speedup 1.0000x reference)
import functools

import jax
import jax.numpy as jnp
from jax import lax
from jax.experimental import pallas as pl
from jax.experimental.pallas import tpu as pltpu
from jax.experimental.pallas import tpu_sc as plsc

VOCAB = 100000
D = 128
B = 16384
K = 10
L = 16
NC, NS = 2, 16
NW = NC * NS
NB = B // NW
C = 32
NCHUNK = NB // C
NG = NCHUNK // 2
NJ = D // L
PP = L + 1

_mesh = plsc.VectorSubcoreMesh(core_axis_name="c", subcore_axis_name="s")


@functools.partial(
    pl.kernel,
    mesh=_mesh,
    compiler_params=pltpu.CompilerParams(needs_layout_passes=False),
    out_type=[
        jax.ShapeDtypeStruct((B,), jnp.float32),
        jax.ShapeDtypeStruct((B,), jnp.float32),
    ],
    scratch_types=[
        pltpu.VMEM((NB,), jnp.int32),
        pltpu.VMEM((NB,), jnp.int32),
        pltpu.VMEM((NB * K,), jnp.int32),
        pltpu.VMEM((K, NB), jnp.int32),
        pltpu.VMEM((C, D), jnp.float32),
        pltpu.VMEM((C, D), jnp.float32),
        pltpu.VMEM((C, D), jnp.float32),
        pltpu.VMEM((C, D), jnp.float32),
        pltpu.VMEM((K * C, D), jnp.float32),
        pltpu.VMEM((K * C, D), jnp.float32),
        pltpu.VMEM((C, PP), jnp.float32),
        pltpu.VMEM((C, PP), jnp.float32),
        pltpu.VMEM((C,), jnp.float32),
        pltpu.VMEM((C,), jnp.float32),
        pltpu.SemaphoreType.DMA,
        pltpu.SemaphoreType.DMA,
    ],
)
def _sc_gather_dot(pos_u_hbm, pos_v_hbm, neg_hbm, target_hbm, context_hbm,
                   pos_out, neg_out,
                   uidx, vidx, nraw, nidx, t0, t1, v0, v1, n0, n1,
                   ppart, npart, psco, nsco, sem0, sem1):
    wid = lax.axis_index("s") * NC + lax.axis_index("c")
    base = pl.multiple_of(wid * NB, NB)
    lanes = lax.broadcasted_iota(jnp.int32, (L,), 0)

    pltpu.sync_copy(pos_u_hbm.at[pl.ds(base, NB)], uidx)
    pltpu.sync_copy(pos_v_hbm.at[pl.ds(base, NB)], vidx)
    pltpu.sync_copy(neg_hbm.at[pl.ds(base * K, NB * K)], nraw)

    def tr_body(g, carry):
        addr = (g * L + lanes) * K
        row0 = pl.multiple_of(g * L, L)
        for k in range(K):
            nidx[k, pl.ds(row0, L)] = plsc.load_gather(nraw, [addr + k])
        return carry

    lax.fori_loop(0, NB // L, tr_body, 0, unroll=False)

    def fire(off, tb, vb, nb, sem):
        pltpu.async_copy(target_hbm.at[uidx.at[pl.ds(off, C)]], tb, sem)
        pltpu.async_copy(context_hbm.at[vidx.at[pl.ds(off, C)]], vb, sem)
        for k in range(K):
            pltpu.async_copy(context_hbm.at[nidx.at[k, pl.ds(off, C)]],
                             nb.at[pl.ds(k * C, C)], sem)

    def drain(tb, vb, nb, sem):
        pltpu.make_async_copy(target_hbm.at[pl.ds(0, C), :], tb, sem).wait()
        pltpu.make_async_copy(context_hbm.at[pl.ds(0, C), :], vb, sem).wait()
        pltpu.make_async_copy(context_hbm.at[pl.ds(0, K * C), :], nb, sem).wait()

    def compute(tb, vb, nb, out_off):
        def b_body(b, carry):
            accp = None
            accn = None
            for j in range(NJ):
                sl = pl.ds(j * L, L)
                t = tb[b, sl]
                v = vb[b, sl]
                ns = nb[b, sl]
                for k in range(1, K):
                    ns = ns + nb[k * C + b, sl]
                if accp is None:
                    accp = t * v
                    accn = t * ns
                else:
                    accp = accp + t * v
                    accn = accn + t * ns
            ppart[b, pl.ds(0, L)] = accp
            npart[b, pl.ds(0, L)] = accn
            return carry

        lax.fori_loop(0, C, b_body, 0, unroll=False)

        def g_body(g, carry):
            rows = g * L + lanes
            sp = None
            sn = None
            for j in range(L):
                col = jnp.full((L,), j, jnp.int32)
                cp = plsc.load_gather(ppart, [rows, col])
                cn = plsc.load_gather(npart, [rows, col])
                sp = cp if sp is None else sp + cp
                sn = cn if sn is None else sn + cn
            row0 = pl.multiple_of(g * L, L)
            psco[pl.ds(row0, L)] = sp
            nsco[pl.ds(row0, L)] = sn
            return carry

        lax.fori_loop(0, C // L, g_body, 0, unroll=False)
        pltpu.sync_copy(psco, pos_out.at[pl.ds(out_off, C)])
        pltpu.sync_copy(nsco, neg_out.at[pl.ds(out_off, C)])

    fire(0, t0, v0, n0, sem0)

    def group(g, carry):
        off0 = pl.multiple_of(g * (2 * C), 2 * C)
        fire(off0 + C, t1, v1, n1, sem1)
        drain(t0, v0, n0, sem0)
        compute(t0, v0, n0, base + off0)
        off2 = pl.multiple_of(
            jnp.minimum(off0 + 2 * C, NB - C).astype(jnp.int32), C)
        fire(off2, t0, v0, n0, sem0)
        drain(t1, v1, n1, sem1)
        compute(t1, v1, n1, base + off0 + C)
        return carry

    lax.fori_loop(0, NG, group, 0, unroll=False)
    drain(t0, v0, n0, sem0)


def _loss_body(p_ref, n_ref, o_ref):
    tot = jnp.sum(jnp.log(jax.nn.sigmoid(p_ref[...]))
                  + jnp.log(jax.nn.sigmoid(-n_ref[...])))
    o_ref[0, 0] = -tot / B


_loss_call = pl.pallas_call(
    _loss_body,
    out_shape=jax.ShapeDtypeStruct((1, 1), jnp.float32),
    out_specs=pl.BlockSpec(memory_space=pltpu.SMEM),
)


@jax.jit
def kernel(pos_u, pos_v, neg_v, target_table, context_table):
    pos_s, neg_s = _sc_gather_dot(pos_u, pos_v, neg_v.reshape(B * K),
                                  target_table, context_table)
    return _loss_call(pos_s.reshape(D, B // D), neg_s.reshape(D, B // D))[0, 0]


# speedup vs baseline: 1.1233x; 1.1233x over previous
import functools

import jax
import jax.numpy as jnp
from jax import lax
from jax.experimental import pallas as pl
from jax.experimental.pallas import tpu as pltpu
from jax.experimental.pallas import tpu_sc as plsc

VOCAB = 100000
D = 128
B = 16384
K = 10
L = 16
NC, NS = 2, 16
NW = NC * NS
NB = B // NW
C = 32
NCHUNK = NB // C
NG = NCHUNK // 2
NJ = D // L

_mesh = plsc.VectorSubcoreMesh(core_axis_name="c", subcore_axis_name="s")


@functools.partial(
    pl.kernel,
    mesh=_mesh,
    compiler_params=pltpu.CompilerParams(needs_layout_passes=False),
    out_type=[
        jax.ShapeDtypeStruct((B,), jnp.float32),
        jax.ShapeDtypeStruct((B,), jnp.float32),
    ],
    scratch_types=[
        pltpu.VMEM((NB,), jnp.int32),
        pltpu.VMEM((NB,), jnp.int32),
        pltpu.VMEM((K, NB), jnp.int32),
        pltpu.VMEM((C, D), jnp.float32),
        pltpu.VMEM((C, D), jnp.float32),
        pltpu.VMEM((C, D), jnp.float32),
        pltpu.VMEM((C, D), jnp.float32),
        pltpu.VMEM((K * C, D), jnp.float32),
        pltpu.VMEM((K * C, D), jnp.float32),
        pltpu.VMEM((C, L), jnp.float32),
        pltpu.VMEM((C, L), jnp.float32),
        pltpu.VMEM((C,), jnp.float32),
        pltpu.VMEM((C,), jnp.float32),
        pltpu.SemaphoreType.DMA,
        pltpu.SemaphoreType.DMA,
    ],
)
def _sc_gather_dot(pos_u_hbm, pos_v_hbm, negT_hbm, target_hbm, context_hbm,
                   pos_out, neg_out,
                   uidx, vidx, nidx, t0, t1, v0, v1, n0, n1,
                   ppart, npart, psco, nsco, sem0, sem1):
    wid = lax.axis_index("s") * NC + lax.axis_index("c")
    base = pl.multiple_of(wid * NB, NB)
    lanes = lax.broadcasted_iota(jnp.int32, (L,), 0)

    pltpu.sync_copy(pos_u_hbm.at[pl.ds(base, NB)], uidx)
    pltpu.sync_copy(pos_v_hbm.at[pl.ds(base, NB)], vidx)
    for k in range(K):
        pltpu.sync_copy(negT_hbm.at[k, pl.ds(base, NB)], nidx.at[k])

    def fire(off, tb, vb, nb, sem):
        pltpu.async_copy(target_hbm.at[uidx.at[pl.ds(off, C)]], tb, sem)
        pltpu.async_copy(context_hbm.at[vidx.at[pl.ds(off, C)]], vb, sem)
        for k in range(K):
            pltpu.async_copy(context_hbm.at[nidx.at[k, pl.ds(off, C)]],
                             nb.at[pl.ds(k * C, C)], sem)

    def drain(tb, vb, nb, sem):
        pltpu.make_async_copy(target_hbm.at[pl.ds(0, C), :], tb, sem).wait()
        pltpu.make_async_copy(context_hbm.at[pl.ds(0, C), :], vb, sem).wait()
        pltpu.make_async_copy(context_hbm.at[pl.ds(0, K * C), :], nb, sem).wait()

    def compute(tb, vb, nb, out_off):
        def b_body(b, carry):
            accp = None
            accn = None
            for j in range(NJ):
                sl = pl.ds(j * L, L)
                t = tb[b, sl]
                v = vb[b, sl]
                ns = nb[b, sl]
                for k in range(1, K):
                    ns = ns + nb[k * C + b, sl]
                if accp is None:
                    accp = t * v
                    accn = t * ns
                else:
                    accp = accp + t * v
                    accn = accn + t * ns
            ppart[b, :] = accp
            npart[b, :] = accn
            return carry

        lax.fori_loop(0, C, b_body, 0, unroll=False)

        def g_body(g, carry):
            rows = g * L + lanes
            sp = None
            sn = None
            for j in range(L):
                col = jnp.full((L,), j, jnp.int32)
                cp = plsc.load_gather(ppart, [rows, col])
                cn = plsc.load_gather(npart, [rows, col])
                sp = cp if sp is None else sp + cp
                sn = cn if sn is None else sn + cn
            row0 = pl.multiple_of(g * L, L)
            psco[pl.ds(row0, L)] = sp
            nsco[pl.ds(row0, L)] = sn
            return carry

        lax.fori_loop(0, C // L, g_body, 0, unroll=False)
        pltpu.sync_copy(psco, pos_out.at[pl.ds(out_off, C)])
        pltpu.sync_copy(nsco, neg_out.at[pl.ds(out_off, C)])

    fire(0, t0, v0, n0, sem0)

    def group(g, carry):
        off0 = pl.multiple_of(g * (2 * C), 2 * C)
        fire(off0 + C, t1, v1, n1, sem1)
        drain(t0, v0, n0, sem0)
        compute(t0, v0, n0, base + off0)
        off2 = pl.multiple_of(
            jnp.minimum(off0 + 2 * C, NB - C).astype(jnp.int32), C)
        fire(off2, t0, v0, n0, sem0)
        drain(t1, v1, n1, sem1)
        compute(t1, v1, n1, base + off0 + C)
        return carry

    lax.fori_loop(0, NG, group, 0, unroll=False)
    drain(t0, v0, n0, sem0)


def _loss_body(p_ref, n_ref, o_ref):
    tot = jnp.sum(jnp.log(jax.nn.sigmoid(p_ref[...]))
                  + jnp.log(jax.nn.sigmoid(-n_ref[...])))
    o_ref[0, 0] = -tot / B


_loss_call = pl.pallas_call(
    _loss_body,
    out_shape=jax.ShapeDtypeStruct((1, 1), jnp.float32),
    out_specs=pl.BlockSpec(memory_space=pltpu.SMEM),
)


@jax.jit
def kernel(pos_u, pos_v, neg_v, target_table, context_table):
    negT = jnp.transpose(neg_v)
    pos_s, neg_s = _sc_gather_dot(pos_u, pos_v, negT,
                                  target_table, context_table)
    return _loss_call(pos_s.reshape(D, B // D), neg_s.reshape(D, B // D))[0, 0]


# speedup vs baseline: 1.1337x; 1.0092x over previous
import functools

import jax
import jax.numpy as jnp
from jax import lax
from jax.experimental import pallas as pl
from jax.experimental.pallas import tpu as pltpu
from jax.experimental.pallas import tpu_sc as plsc

VOCAB = 100000
D = 128
B = 16384
K = 10
L = 16
NC, NS = 2, 16
NW = NC * NS
NB = B // NW
C = 32
NCHUNK = NB // C
NG = NCHUNK // 2
NJ = D // L

_mesh = plsc.VectorSubcoreMesh(core_axis_name="c", subcore_axis_name="s")


@functools.partial(
    pl.kernel,
    mesh=_mesh,
    compiler_params=pltpu.CompilerParams(needs_layout_passes=False,
                                         use_tc_tiling_on_sc=False),
    out_type=[
        jax.ShapeDtypeStruct((B,), jnp.float32),
        jax.ShapeDtypeStruct((B,), jnp.float32),
    ],
    scratch_types=[
        pltpu.VMEM((NB,), jnp.int32),
        pltpu.VMEM((NB,), jnp.int32),
        pltpu.VMEM((K, NB), jnp.int32),
        pltpu.VMEM((C, D), jnp.float32),
        pltpu.VMEM((C, D), jnp.float32),
        pltpu.VMEM((C, D), jnp.float32),
        pltpu.VMEM((C, D), jnp.float32),
        pltpu.VMEM((K * C, D), jnp.float32),
        pltpu.VMEM((K * C, D), jnp.float32),
        pltpu.VMEM((C, L), jnp.float32),
        pltpu.VMEM((C, L), jnp.float32),
        pltpu.VMEM((C,), jnp.float32),
        pltpu.VMEM((C,), jnp.float32),
        pltpu.SemaphoreType.DMA,
        pltpu.SemaphoreType.DMA,
    ],
)
def _sc_gather_dot(pos_u_hbm, pos_v_hbm, negT_hbm, target_hbm, context_hbm,
                   pos_out, neg_out,
                   uidx, vidx, nidx, t0, t1, v0, v1, n0, n1,
                   ppart, npart, psco, nsco, sem0, sem1):
    wid = lax.axis_index("s") * NC + lax.axis_index("c")
    base = pl.multiple_of(wid * NB, NB)
    lanes = lax.broadcasted_iota(jnp.int32, (L,), 0)

    pltpu.sync_copy(pos_u_hbm.at[pl.ds(base, NB)], uidx)
    pltpu.sync_copy(pos_v_hbm.at[pl.ds(base, NB)], vidx)
    for k in range(K):
        pltpu.sync_copy(negT_hbm.at[k, pl.ds(base, NB)], nidx.at[k])

    def fire(off, tb, vb, nb, sem):
        pltpu.async_copy(target_hbm.at[uidx.at[pl.ds(off, C)]], tb, sem)
        pltpu.async_copy(context_hbm.at[vidx.at[pl.ds(off, C)]], vb, sem)
        for k in range(K):
            pltpu.async_copy(context_hbm.at[nidx.at[k, pl.ds(off, C)]],
                             nb.at[pl.ds(k * C, C)], sem)

    def drain(tb, vb, nb, sem):
        pltpu.make_async_copy(target_hbm.at[pl.ds(0, C), :], tb, sem).wait()
        pltpu.make_async_copy(context_hbm.at[pl.ds(0, C), :], vb, sem).wait()
        pltpu.make_async_copy(context_hbm.at[pl.ds(0, K * C), :], nb, sem).wait()

    def compute(tb, vb, nb, out_off):
        def b_body(b, carry):
            accp = None
            accn = None
            for j in range(NJ):
                sl = pl.ds(j * L, L)
                t = tb[b, sl]
                v = vb[b, sl]
                ns = nb[b, sl]
                for k in range(1, K):
                    ns = ns + nb[k * C + b, sl]
                if accp is None:
                    accp = t * v
                    accn = t * ns
                else:
                    accp = accp + t * v
                    accn = accn + t * ns
            ppart[b, :] = accp
            npart[b, :] = accn
            return carry

        lax.fori_loop(0, C, b_body, 0, unroll=False)

        def g_body(g, carry):
            rows = g * L + lanes
            sp = None
            sn = None
            for j in range(L):
                col = jnp.full((L,), j, jnp.int32)
                cp = plsc.load_gather(ppart, [rows, col])
                cn = plsc.load_gather(npart, [rows, col])
                sp = cp if sp is None else sp + cp
                sn = cn if sn is None else sn + cn
            row0 = pl.multiple_of(g * L, L)
            psco[pl.ds(row0, L)] = sp
            nsco[pl.ds(row0, L)] = sn
            return carry

        lax.fori_loop(0, C // L, g_body, 0, unroll=False)
        pltpu.sync_copy(psco, pos_out.at[pl.ds(out_off, C)])
        pltpu.sync_copy(nsco, neg_out.at[pl.ds(out_off, C)])

    fire(0, t0, v0, n0, sem0)

    def group(g, carry):
        off0 = pl.multiple_of(g * (2 * C), 2 * C)
        fire(off0 + C, t1, v1, n1, sem1)
        drain(t0, v0, n0, sem0)
        compute(t0, v0, n0, base + off0)
        off2 = pl.multiple_of(
            jnp.minimum(off0 + 2 * C, NB - C).astype(jnp.int32), C)
        fire(off2, t0, v0, n0, sem0)
        drain(t1, v1, n1, sem1)
        compute(t1, v1, n1, base + off0 + C)
        return carry

    lax.fori_loop(0, NG, group, 0, unroll=False)
    drain(t0, v0, n0, sem0)


def _loss_body(p_ref, n_ref, o_ref):
    tot = jnp.sum(jnp.log(jax.nn.sigmoid(p_ref[...]))
                  + jnp.log(jax.nn.sigmoid(-n_ref[...])))
    o_ref[0, 0] = -tot / B


_loss_call = pl.pallas_call(
    _loss_body,
    out_shape=jax.ShapeDtypeStruct((1, 1), jnp.float32),
    out_specs=pl.BlockSpec(memory_space=pltpu.SMEM),
)


@jax.jit
def kernel(pos_u, pos_v, neg_v, target_table, context_table):
    negT = jnp.transpose(neg_v)
    pos_s, neg_s = _sc_gather_dot(pos_u, pos_v, negT,
                                  target_table, context_table)
    return _loss_call(pos_s.reshape(D, B // D), neg_s.reshape(D, B // D))[0, 0]
